# Initial kernel scaffold; baseline (speedup 1.0000x reference)
import jax
import jax.numpy as jnp
from jax.experimental import pallas as pl


def kernel(y_pilots, freq_idx, freq_w, time_idx, time_w):
    raise NotImplementedError("write your pallas kernel here")


# speedup vs baseline: 1.7996x; 1.7996x over previous
import functools

import jax
import jax.numpy as jnp
from jax import lax
from jax.experimental import pallas as pl
from jax.experimental.pallas import tpu as pltpu
from jax.experimental.pallas import tpu_sc as plsc

_L = 16


def _sc_freq_interp(y2, fidx_p, fw_p, n_workers, rows_per_w):
    R, P = y2.shape
    K, S_PAD = fidx_p.shape
    n_chunks = S_PAD // _L
    mesh = plsc.VectorSubcoreMesh(core_axis_name="c", subcore_axis_name="s",
                                  num_cores=2, num_subcores=16)
    y_flat = y2.reshape(R * P)
    fidx_flat = fidx_p.reshape(K * S_PAD)
    fw_flat = fw_p.reshape(K * S_PAD)

    @functools.partial(
        pl.kernel,
        out_type=jax.ShapeDtypeStruct((R * S_PAD,), jnp.float32),
        mesh=mesh,
        compiler_params=pltpu.CompilerParams(needs_layout_passes=False),
        scratch_types=[
            pltpu.VMEM((rows_per_w * P,), jnp.float32),
            pltpu.VMEM((K * S_PAD,), jnp.int32),
            pltpu.VMEM((K * S_PAD,), jnp.float32),
            pltpu.VMEM((rows_per_w * S_PAD,), jnp.float32),
        ],
    )
    def sc_k(y_hbm, fidx_hbm, fw_hbm, out_hbm, y_v, idx_v, w_v, out_v):
        wid = lax.axis_index("s") * 2 + lax.axis_index("c")
        base = wid * rows_per_w
        pltpu.sync_copy(y_hbm.at[pl.ds(base * P, rows_per_w * P)], y_v)
        pltpu.sync_copy(fidx_hbm, idx_v)
        pltpu.sync_copy(fw_hbm, w_v)

        def body(j, carry):
            col = j * _L
            iv = [idx_v[pl.ds(k * S_PAD + col, _L)] for k in range(K)]
            wv = [w_v[pl.ds(k * S_PAD + col, _L)] for k in range(K)]
            for r in range(rows_per_w):
                acc = plsc.load_gather(y_v, [iv[0] + r * P]) * wv[0]
                for k in range(1, K):
                    acc = acc + plsc.load_gather(y_v, [iv[k] + r * P]) * wv[k]
                out_v[pl.ds(r * S_PAD + col, _L)] = acc
            return carry

        lax.fori_loop(0, n_chunks, body, 0)
        pltpu.sync_copy(out_v, out_hbm.at[pl.ds(base * S_PAD, rows_per_w * S_PAD)])

    return sc_k(y_flat, fidx_flat, fw_flat).reshape(R, S_PAD)


def _tc_time_combine(hf, time_idx, time_w, qb):
    RT, S_PAD = hf.shape
    K, O = time_idx.shape
    T = 3
    NQ = RT // T
    rows, cols = O * qb, T * qb

    def tc_body(tidx_ref, tw_ref, hf_ref, out_ref):
        r_iota = lax.broadcasted_iota(jnp.int32, (rows, cols), 0)
        c_iota = lax.broadcasted_iota(jnp.int32, (rows, cols), 1)
        qq_eq = (r_iota // O) == (c_iota // T)
        t_col = (c_iota % T).astype(jnp.float32)
        ro = (lax.broadcasted_iota(jnp.int32, (rows, O), 0) % O
              == lax.broadcasted_iota(jnp.int32, (rows, O), 1))
        rof = ro.astype(jnp.float32)
        tw = tw_ref[...]
        tidx_f = tidx_ref[...].astype(jnp.float32)
        dn = (((1,), (1,)), ((), ()))
        a_kw = lax.dot_general(rof, tw, dn,
                               preferred_element_type=jnp.float32)
        b_ki = lax.dot_general(rof, tidx_f, dn,
                               preferred_element_type=jnp.float32)
        cbig = jnp.zeros((rows, cols), jnp.float32)
        for k in range(K):
            sel = qq_eq & (b_ki[:, k:k + 1] == t_col)
            cbig = cbig + jnp.where(sel, a_kw[:, k:k + 1], 0.0)
        out_ref[...] = lax.dot(cbig, hf_ref[...],
                               precision=lax.Precision.HIGHEST,
                               preferred_element_type=jnp.float32)

    return pl.pallas_call(
        tc_body,
        grid=(NQ // qb,),
        in_specs=[
            pl.BlockSpec((K, O), lambda i: (0, 0)),
            pl.BlockSpec((K, O), lambda i: (0, 0)),
            pl.BlockSpec((T * qb, S_PAD), lambda i: (i, 0)),
        ],
        out_specs=pl.BlockSpec((O * qb, S_PAD), lambda i: (i, 0)),
        out_shape=jax.ShapeDtypeStruct((NQ * O, S_PAD), jnp.float32),
    )(time_idx, time_w, hf)


def kernel(y_pilots, freq_idx, freq_w, time_idx, time_w):
    B, A, T, P = y_pilots.shape
    K, NSC = freq_idx.shape
    O = time_idx.shape[1]
    S_PAD = ((NSC + 2 * _L - 1) // (2 * _L)) * (2 * _L)
    R = B * A * T
    n_workers = 32
    rows_per_w = R // n_workers

    y2 = y_pilots.reshape(R, P)
    fidx_p = jnp.zeros((K, S_PAD), jnp.int32).at[:, :NSC].set(freq_idx)
    fw_p = jnp.zeros((K, S_PAD), jnp.float32).at[:, :NSC].set(freq_w)

    hf = _sc_freq_interp(y2, fidx_p, fw_p, n_workers, rows_per_w)
    out = _tc_time_combine(hf, time_idx, time_w, qb=8)
    return out.reshape(B, A, O, S_PAD)[..., :NSC]


# speedup vs baseline: 1.9582x; 1.0881x over previous
import functools

import jax
import jax.numpy as jnp
from jax import lax
from jax.experimental import pallas as pl
from jax.experimental.pallas import tpu as pltpu
from jax.experimental.pallas import tpu_sc as plsc

_L = 16


def _sc_freq_interp(y2, fidx_p, fw_p, n_workers, rows_per_w):
    R, P = y2.shape
    K, S_PAD = fidx_p.shape
    n_chunks = S_PAD // _L
    mesh = plsc.VectorSubcoreMesh(core_axis_name="c", subcore_axis_name="s",
                                  num_cores=2, num_subcores=16)
    @functools.partial(
        pl.kernel,
        out_type=jax.ShapeDtypeStruct((R, S_PAD), jnp.float32),
        mesh=mesh,
        compiler_params=pltpu.CompilerParams(needs_layout_passes=False),
        scratch_types=[
            pltpu.VMEM((rows_per_w * P,), jnp.float32),
            pltpu.VMEM((K, S_PAD), jnp.int32),
            pltpu.VMEM((K, S_PAD), jnp.float32),
            pltpu.VMEM((rows_per_w, S_PAD), jnp.float32),
        ],
    )
    def sc_k(y_hbm, fidx_hbm, fw_hbm, out_hbm, y_v, idx_v, w_v, out_v):
        wid = lax.axis_index("s") * 2 + lax.axis_index("c")
        base = wid * rows_per_w
        pltpu.sync_copy(y_hbm.at[pl.ds(base * P, rows_per_w * P)], y_v)
        pltpu.sync_copy(fidx_hbm, idx_v)
        pltpu.sync_copy(fw_hbm, w_v)

        def body(j, carry):
            col = j * _L
            iv = [idx_v[k, pl.ds(col, _L)] for k in range(K)]
            wv = [w_v[k, pl.ds(col, _L)] for k in range(K)]
            for r in range(rows_per_w):
                acc = plsc.load_gather(y_v, [iv[0] + r * P]) * wv[0]
                for k in range(1, K):
                    acc = acc + plsc.load_gather(y_v, [iv[k] + r * P]) * wv[k]
                out_v[r, pl.ds(col, _L)] = acc
            return carry

        lax.fori_loop(0, n_chunks, body, 0)
        pltpu.sync_copy(out_v, out_hbm.at[pl.ds(base, rows_per_w)])

    return sc_k(y2.reshape(R * P), fidx_p, fw_p)


def _tc_time_combine(hf, time_idx, time_w, qb, nsc):
    RT, S_PAD = hf.shape
    K, O = time_idx.shape
    T = 3
    NQ = RT // T
    rows, cols = O * qb, T * qb

    def tc_body(tidx_ref, tw_ref, hf_ref, out_ref):
        r_iota = lax.broadcasted_iota(jnp.int32, (rows, cols), 0)
        c_iota = lax.broadcasted_iota(jnp.int32, (rows, cols), 1)
        qq_eq = (r_iota // O) == (c_iota // T)
        t_col = (c_iota % T).astype(jnp.float32)
        ro = (lax.broadcasted_iota(jnp.int32, (rows, O), 0) % O
              == lax.broadcasted_iota(jnp.int32, (rows, O), 1))
        rof = ro.astype(jnp.float32)
        tw = tw_ref[...]
        tidx_f = tidx_ref[...].astype(jnp.float32)
        dn = (((1,), (1,)), ((), ()))
        a_kw = lax.dot_general(rof, tw, dn,
                               preferred_element_type=jnp.float32)
        b_ki = lax.dot_general(rof, tidx_f, dn,
                               preferred_element_type=jnp.float32)
        cbig = jnp.zeros((rows, cols), jnp.float32)
        for k in range(K):
            sel = qq_eq & (b_ki[:, k:k + 1] == t_col)
            cbig = cbig + jnp.where(sel, a_kw[:, k:k + 1], 0.0)
        out_ref[...] = lax.dot(cbig, hf_ref[:, :nsc],
                               precision=lax.Precision.HIGHEST,
                               preferred_element_type=jnp.float32)

    return pl.pallas_call(
        tc_body,
        grid=(NQ // qb,),
        in_specs=[
            pl.BlockSpec((K, O), lambda i: (0, 0)),
            pl.BlockSpec((K, O), lambda i: (0, 0)),
            pl.BlockSpec((T * qb, S_PAD), lambda i: (i, 0)),
        ],
        out_specs=pl.BlockSpec((O * qb, nsc), lambda i: (i, 0)),
        out_shape=jax.ShapeDtypeStruct((NQ * O, nsc), jnp.float32),
    )(time_idx, time_w, hf)


def kernel(y_pilots, freq_idx, freq_w, time_idx, time_w):
    B, A, T, P = y_pilots.shape
    K, NSC = freq_idx.shape
    O = time_idx.shape[1]
    S_PAD = ((NSC + 2 * _L - 1) // (2 * _L)) * (2 * _L)
    R = B * A * T
    n_workers = 32
    rows_per_w = R // n_workers

    y2 = y_pilots.reshape(R, P)
    fidx_p = jnp.zeros((K, S_PAD), jnp.int32).at[:, :NSC].set(freq_idx)
    fw_p = jnp.zeros((K, S_PAD), jnp.float32).at[:, :NSC].set(freq_w)

    hf = _sc_freq_interp(y2, fidx_p, fw_p, n_workers, rows_per_w)
    out = _tc_time_combine(hf, time_idx, time_w, qb=32, nsc=NSC)
    return out.reshape(B, A, O, NSC)


# speedup vs baseline: 4.1459x; 2.1172x over previous
import functools

import jax
import jax.numpy as jnp
from jax import lax
from jax.experimental import pallas as pl
from jax.experimental.pallas import tpu as pltpu
from jax.experimental.pallas import tpu_sc as plsc

_L = 16


def _sc_freq_interp(y2, fidx_p, fw_p, nb, na):
    R, P = y2.shape
    K, S_PAD = fidx_p.shape
    n_chunks = S_PAD // _L
    n_workers = 32
    rows_per_w = R // n_workers
    b_per_w = rows_per_w // (na * K)
    mesh = plsc.VectorSubcoreMesh(core_axis_name="c", subcore_axis_name="s",
                                  num_cores=2, num_subcores=16)

    @functools.partial(
        pl.kernel,
        out_type=jax.ShapeDtypeStruct((K, nb, na, S_PAD), jnp.float32),
        mesh=mesh,
        compiler_params=pltpu.CompilerParams(needs_layout_passes=False),
        scratch_types=[
            pltpu.VMEM((rows_per_w * P,), jnp.float32),
            pltpu.VMEM((K, S_PAD), jnp.int32),
            pltpu.VMEM((K, S_PAD), jnp.float32),
            pltpu.VMEM((K, b_per_w, na, S_PAD), jnp.float32),
        ],
    )
    def sc_k(y_hbm, fidx_hbm, fw_hbm, out_hbm, y_v, idx_v, w_v, out_v):
        wid = lax.axis_index("s") * 2 + lax.axis_index("c")
        base = wid * rows_per_w
        pltpu.sync_copy(y_hbm.at[pl.ds(base * P, rows_per_w * P)], y_v)
        pltpu.sync_copy(fidx_hbm, idx_v)
        pltpu.sync_copy(fw_hbm, w_v)

        def body(j, carry):
            col = j * _L
            iv = [idx_v[k, pl.ds(col, _L)] for k in range(K)]
            wv = [w_v[k, pl.ds(col, _L)] for k in range(K)]
            for r in range(rows_per_w):
                t = r % 3
                bl = (r // 3) // na
                a = (r // 3) % na
                acc = plsc.load_gather(y_v, [iv[0] + r * P]) * wv[0]
                for k in range(1, K):
                    acc = acc + plsc.load_gather(y_v, [iv[k] + r * P]) * wv[k]
                out_v[t, bl, a, pl.ds(col, _L)] = acc
            return carry

        lax.fori_loop(0, n_chunks, body, 0)
        for t in range(K):
            for bl in range(b_per_w):
                pltpu.sync_copy(out_v.at[t, bl],
                                out_hbm.at[t, wid * b_per_w + bl])

    return sc_k(y2.reshape(R * P), fidx_p, fw_p)


def _tc_time_combine(hf4, time_idx, time_w, bb, nsc):
    T, NB, NA, S_PAD = hf4.shape
    K, O = time_idx.shape

    def tc_body(tidx_ref, tw_ref, hf_ref, out_ref):
        h = [hf_ref[t, :, :, :nsc] for t in range(T)]
        for o in range(O):
            c = []
            for t in range(T):
                ct = jnp.float32(0.0)
                for k in range(K):
                    ct = ct + jnp.where(tidx_ref[k, o] == t,
                                        tw_ref[k, o], jnp.float32(0.0))
                c.append(ct)
            acc = h[0] * c[0]
            for t in range(1, T):
                acc = acc + h[t] * c[t]
            out_ref[:, o, :, :] = acc

    return pl.pallas_call(
        tc_body,
        grid=(NB // bb,),
        in_specs=[
            pl.BlockSpec(memory_space=pltpu.SMEM),
            pl.BlockSpec(memory_space=pltpu.SMEM),
            pl.BlockSpec((T, bb, NA, S_PAD), lambda i: (0, i, 0, 0)),
        ],
        out_specs=pl.BlockSpec((bb, O, NA, nsc), lambda i: (i, 0, 0, 0)),
        out_shape=jax.ShapeDtypeStruct((NB, O, NA, nsc), jnp.float32),
    )(time_idx, time_w, hf4)


def kernel(y_pilots, freq_idx, freq_w, time_idx, time_w):
    B, A, T, P = y_pilots.shape
    K, NSC = freq_idx.shape
    O = time_idx.shape[1]
    S_PAD = ((NSC + 127) // 128) * 128
    R = B * A * T

    y2 = y_pilots.reshape(R, P)
    fidx_p = jnp.zeros((K, S_PAD), jnp.int32).at[:, :NSC].set(freq_idx)
    fw_p = jnp.zeros((K, S_PAD), jnp.float32).at[:, :NSC].set(freq_w)

    hf4 = _sc_freq_interp(y2, fidx_p, fw_p, B, A)
    out4 = _tc_time_combine(hf4, time_idx, time_w, bb=8, nsc=NSC)
    return jnp.transpose(out4, (0, 2, 1, 3))


# speedup vs baseline: 5.4739x; 1.3203x over previous
import functools

import jax
import jax.numpy as jnp
from jax import lax
from jax.experimental import pallas as pl
from jax.experimental.pallas import tpu as pltpu
from jax.experimental.pallas import tpu_sc as plsc

_L = 16


def _sc_freq_interp(y2, fidx_p, fw_p, nb, na):
    R, P = y2.shape
    P_PAD = ((P + 7) // 8) * 8
    y2 = jnp.pad(y2, ((0, 0), (0, P_PAD - P)))
    P = P_PAD
    K, S_PAD = fidx_p.shape
    n_chunks = S_PAD // _L
    n_workers = 32
    rows_per_w = R // n_workers
    b_per_w = rows_per_w // (na * K)
    mesh = plsc.VectorSubcoreMesh(core_axis_name="c", subcore_axis_name="s",
                                  num_cores=2, num_subcores=16)

    @functools.partial(
        pl.kernel,
        out_type=jax.ShapeDtypeStruct((K, nb, na, S_PAD), jnp.float32),
        mesh=mesh,
        compiler_params=pltpu.CompilerParams(needs_layout_passes=False),
        scratch_types=[
            pltpu.VMEM((rows_per_w * P,), jnp.float32),
            pltpu.VMEM((K, S_PAD), jnp.int32),
            pltpu.VMEM((K, S_PAD), jnp.float32),
            pltpu.VMEM((K, b_per_w, na, S_PAD), jnp.float32),
        ],
    )
    def sc_k(y_hbm, fidx_hbm, fw_hbm, out_hbm, y_v, idx_v, w_v, out_v):
        wid = lax.axis_index("s") * 2 + lax.axis_index("c")
        base = wid * rows_per_w
        pltpu.sync_copy(y_hbm.at[pl.ds(base * P, rows_per_w * P)], y_v)
        pltpu.sync_copy(fidx_hbm, idx_v)
        pltpu.sync_copy(fw_hbm, w_v)

        @plsc.parallel_loop(0, n_chunks)
        def body(j):
            col = j * _L
            iv = [idx_v[k, pl.ds(col, _L)] for k in range(K)]
            wv = [w_v[k, pl.ds(col, _L)] for k in range(K)]
            for r in range(rows_per_w):
                t = r % 3
                bl = (r // 3) // na
                a = (r // 3) % na
                row = y_v.at[pl.ds(r * P, P)]
                acc = plsc.load_gather(row, [iv[0]]) * wv[0]
                for k in range(1, K):
                    acc = acc + plsc.load_gather(row, [iv[k]]) * wv[k]
                out_v[t, bl, a, pl.ds(col, _L)] = acc
        for t in range(K):
            for bl in range(b_per_w):
                pltpu.sync_copy(out_v.at[t, bl],
                                out_hbm.at[t, wid * b_per_w + bl])

    return sc_k(y2.reshape(R * P), fidx_p, fw_p)


def _tc_time_combine(hf4, time_idx, time_w, bb, nsc):
    T, NB, NA, S_PAD = hf4.shape
    K, O = time_idx.shape

    def tc_body(tidx_ref, tw_ref, hf_ref, out_ref):
        h = [hf_ref[t, :, :, :nsc] for t in range(T)]
        for o in range(O):
            c = []
            for t in range(T):
                ct = jnp.float32(0.0)
                for k in range(K):
                    ct = ct + jnp.where(tidx_ref[k, o] == t,
                                        tw_ref[k, o], jnp.float32(0.0))
                c.append(ct)
            acc = h[0] * c[0]
            for t in range(1, T):
                acc = acc + h[t] * c[t]
            out_ref[:, o, :, :] = acc

    return pl.pallas_call(
        tc_body,
        grid=(NB // bb,),
        in_specs=[
            pl.BlockSpec(memory_space=pltpu.SMEM),
            pl.BlockSpec(memory_space=pltpu.SMEM),
            pl.BlockSpec((T, bb, NA, S_PAD), lambda i: (0, i, 0, 0)),
        ],
        out_specs=pl.BlockSpec((bb, O, NA, nsc), lambda i: (i, 0, 0, 0)),
        out_shape=jax.ShapeDtypeStruct((NB, O, NA, nsc), jnp.float32),
    )(time_idx, time_w, hf4)


def kernel(y_pilots, freq_idx, freq_w, time_idx, time_w):
    B, A, T, P = y_pilots.shape
    K, NSC = freq_idx.shape
    O = time_idx.shape[1]
    S_PAD = ((NSC + 127) // 128) * 128
    R = B * A * T

    y2 = y_pilots.reshape(R, P)
    fidx_p = jnp.zeros((K, S_PAD), jnp.int32).at[:, :NSC].set(freq_idx)
    fw_p = jnp.zeros((K, S_PAD), jnp.float32).at[:, :NSC].set(freq_w)

    hf4 = _sc_freq_interp(y2, fidx_p, fw_p, B, A)
    out4 = _tc_time_combine(hf4, time_idx, time_w, bb=8, nsc=NSC)
    return jnp.transpose(out4, (0, 2, 1, 3))


# speedup vs baseline: 5.5160x; 1.0077x over previous
import functools

import jax
import jax.numpy as jnp
from jax import lax
from jax.experimental import pallas as pl
from jax.experimental.pallas import tpu as pltpu
from jax.experimental.pallas import tpu_sc as plsc

_L = 16


def _sc_freq_interp(y2, fidx_p, fw_p, nb, na):
    R, P = y2.shape
    P_PAD = ((P + 7) // 8) * 8
    y2 = jnp.pad(y2, ((0, 0), (0, P_PAD - P)))
    P = P_PAD
    K, S_PAD = fidx_p.shape
    n_chunks = S_PAD // _L
    n_workers = 32
    rows_per_w = R // n_workers
    b_per_w = rows_per_w // (na * K)
    mesh = plsc.VectorSubcoreMesh(core_axis_name="c", subcore_axis_name="s",
                                  num_cores=2, num_subcores=16)

    @functools.partial(
        pl.kernel,
        out_type=jax.ShapeDtypeStruct((K, nb, na, S_PAD), jnp.float32),
        mesh=mesh,
        compiler_params=pltpu.CompilerParams(needs_layout_passes=False),
        scratch_types=[
            pltpu.VMEM((rows_per_w * P,), jnp.float32),
            pltpu.VMEM((K, S_PAD), jnp.int32),
            pltpu.VMEM((K, S_PAD), jnp.float32),
            pltpu.VMEM((K, b_per_w, na, S_PAD), jnp.float32),
            pltpu.SemaphoreType.DMA,
        ],
    )
    def sc_k(y_hbm, fidx_hbm, fw_hbm, out_hbm, y_v, idx_v, w_v, out_v, sem):
        wid = lax.axis_index("s") * 2 + lax.axis_index("c")
        base = wid * rows_per_w
        pltpu.sync_copy(y_hbm.at[pl.ds(base * P, rows_per_w * P)], y_v)
        pltpu.sync_copy(fidx_hbm, idx_v)
        pltpu.sync_copy(fw_hbm, w_v)

        @plsc.parallel_loop(0, n_chunks)
        def body(j):
            col = j * _L
            iv = [idx_v[k, pl.ds(col, _L)] for k in range(K)]
            wv = [w_v[k, pl.ds(col, _L)] for k in range(K)]
            for r in range(rows_per_w):
                t = r % 3
                bl = (r // 3) // na
                a = (r // 3) % na
                row = y_v.at[pl.ds(r * P, P)]
                acc = plsc.load_gather(row, [iv[0]]) * wv[0]
                for k in range(1, K):
                    acc = acc + plsc.load_gather(row, [iv[k]]) * wv[k]
                out_v[t, bl, a, pl.ds(col, _L)] = acc
        cps = [pltpu.async_copy(out_v.at[t, bl],
                                out_hbm.at[t, wid * b_per_w + bl], sem)
               for t in range(K) for bl in range(b_per_w)]
        for cp in cps:
            cp.wait()

    return sc_k(y2.reshape(R * P), fidx_p, fw_p)


def _tc_time_combine(hf4, time_idx, time_w, bb, nsc):
    T, NB, NA, S_PAD = hf4.shape
    K, O = time_idx.shape

    def tc_body(tidx_ref, tw_ref, hf_ref, out_ref):
        h = [hf_ref[t, :, :, :nsc] for t in range(T)]
        for o in range(O):
            c = []
            for t in range(T):
                ct = jnp.float32(0.0)
                for k in range(K):
                    ct = ct + jnp.where(tidx_ref[k, o] == t,
                                        tw_ref[k, o], jnp.float32(0.0))
                c.append(ct)
            acc = h[0] * c[0]
            for t in range(1, T):
                acc = acc + h[t] * c[t]
            out_ref[:, o, :, :] = acc

    return pl.pallas_call(
        tc_body,
        grid=(NB // bb,),
        in_specs=[
            pl.BlockSpec(memory_space=pltpu.SMEM),
            pl.BlockSpec(memory_space=pltpu.SMEM),
            pl.BlockSpec((T, bb, NA, S_PAD), lambda i: (0, i, 0, 0)),
        ],
        out_specs=pl.BlockSpec((bb, O, NA, nsc), lambda i: (i, 0, 0, 0)),
        out_shape=jax.ShapeDtypeStruct((NB, O, NA, nsc), jnp.float32),
    )(time_idx, time_w, hf4)


def kernel(y_pilots, freq_idx, freq_w, time_idx, time_w):
    B, A, T, P = y_pilots.shape
    K, NSC = freq_idx.shape
    O = time_idx.shape[1]
    S_PAD = ((NSC + 127) // 128) * 128
    R = B * A * T

    fidx_p = jnp.zeros((K, S_PAD), jnp.int32).at[:, :NSC].set(freq_idx)
    fw_p = jnp.zeros((K, S_PAD), jnp.float32).at[:, :NSC].set(freq_w)

    hf4 = _sc_freq_interp(y_pilots.reshape(R, P), fidx_p, fw_p, B, A)
    out4 = _tc_time_combine(hf4, time_idx, time_w, bb=16, nsc=NSC)
    return jnp.transpose(out4, (0, 2, 1, 3))


# speedup vs baseline: 5.5395x; 1.0043x over previous
import functools

import jax
import jax.numpy as jnp
from jax import lax
from jax.experimental import pallas as pl
from jax.experimental.pallas import tpu as pltpu
from jax.experimental.pallas import tpu_sc as plsc

_L = 16


def _sc_freq_interp(y2, fidx_p, fw_p, nb, na):
    R, P = y2.shape
    K, S_PAD = fidx_p.shape
    n_chunks = S_PAD // _L
    n_workers = 32
    rows_per_w = R // n_workers
    b_per_w = rows_per_w // (na * K)
    mesh = plsc.VectorSubcoreMesh(core_axis_name="c", subcore_axis_name="s",
                                  num_cores=2, num_subcores=16)

    @functools.partial(
        pl.kernel,
        out_type=jax.ShapeDtypeStruct((K, nb, na, S_PAD), jnp.float32),
        mesh=mesh,
        compiler_params=pltpu.CompilerParams(needs_layout_passes=False),
        scratch_types=[
            pltpu.VMEM((rows_per_w * P,), jnp.float32),
            pltpu.VMEM((K, S_PAD), jnp.int32),
            pltpu.VMEM((K, S_PAD), jnp.float32),
            pltpu.VMEM((K, b_per_w, na, S_PAD), jnp.float32),
            pltpu.SemaphoreType.DMA,
        ],
    )
    def sc_k(y_hbm, fidx_hbm, fw_hbm, out_hbm, y_v, idx_v, w_v, out_v, sem):
        wid = lax.axis_index("s") * 2 + lax.axis_index("c")
        base = wid * rows_per_w
        pltpu.sync_copy(y_hbm.at[pl.ds(base * P, rows_per_w * P)], y_v)
        pltpu.sync_copy(fidx_hbm, idx_v)
        pltpu.sync_copy(fw_hbm, w_v)

        @plsc.parallel_loop(0, n_chunks)
        def body(j):
            col = j * _L
            iv = [idx_v[k, pl.ds(col, _L)] for k in range(K)]
            wv = [w_v[k, pl.ds(col, _L)] for k in range(K)]
            for r in range(rows_per_w):
                t = r % 3
                bl = (r // 3) // na
                a = (r // 3) % na
                acc = plsc.load_gather(y_v, [iv[0] + r * P]) * wv[0]
                for k in range(1, K):
                    acc = acc + plsc.load_gather(y_v, [iv[k] + r * P]) * wv[k]
                out_v[t, bl, a, pl.ds(col, _L)] = acc
        cps = [pltpu.async_copy(out_v.at[t, bl],
                                out_hbm.at[t, wid * b_per_w + bl], sem)
               for t in range(K) for bl in range(b_per_w)]
        for cp in cps:
            cp.wait()

    return sc_k(y2.reshape(R * P), fidx_p, fw_p)


def _tc_time_combine(hf4, time_idx, time_w, bb, nsc):
    T, NB, NA, S_PAD = hf4.shape
    K, O = time_idx.shape

    def tc_body(tidx_ref, tw_ref, hf_ref, out_ref):
        h = [hf_ref[t, :, :, :nsc] for t in range(T)]
        for o in range(O):
            c = []
            for t in range(T):
                ct = jnp.float32(0.0)
                for k in range(K):
                    ct = ct + jnp.where(tidx_ref[k, o] == t,
                                        tw_ref[k, o], jnp.float32(0.0))
                c.append(ct)
            acc = h[0] * c[0]
            for t in range(1, T):
                acc = acc + h[t] * c[t]
            out_ref[:, o, :, :] = acc

    return pl.pallas_call(
        tc_body,
        grid=(NB // bb,),
        in_specs=[
            pl.BlockSpec(memory_space=pltpu.SMEM),
            pl.BlockSpec(memory_space=pltpu.SMEM),
            pl.BlockSpec((T, bb, NA, S_PAD), lambda i: (0, i, 0, 0)),
        ],
        out_specs=pl.BlockSpec((bb, O, NA, nsc), lambda i: (i, 0, 0, 0)),
        out_shape=jax.ShapeDtypeStruct((NB, O, NA, nsc), jnp.float32),
    )(time_idx, time_w, hf4)


def kernel(y_pilots, freq_idx, freq_w, time_idx, time_w):
    B, A, T, P = y_pilots.shape
    K, NSC = freq_idx.shape
    O = time_idx.shape[1]
    S_PAD = ((NSC + 127) // 128) * 128
    R = B * A * T

    fidx_p = jnp.zeros((K, S_PAD), jnp.int32).at[:, :NSC].set(freq_idx)
    fw_p = jnp.zeros((K, S_PAD), jnp.float32).at[:, :NSC].set(freq_w)

    hf4 = _sc_freq_interp(y_pilots.reshape(R, P), fidx_p, fw_p, B, A)
    out4 = _tc_time_combine(hf4, time_idx, time_w, bb=4, nsc=NSC)
    return jnp.transpose(out4, (0, 2, 1, 3))


# speedup vs baseline: 6.0031x; 1.0837x over previous
import functools

import jax
import jax.numpy as jnp
from jax import lax
from jax.experimental import pallas as pl
from jax.experimental.pallas import tpu as pltpu
from jax.experimental.pallas import tpu_sc as plsc

_L = 16


def _sc_freq_interp(y2, fidx_p, fw_p, nb, na):
    R, P = y2.shape
    K, S_PAD = fidx_p.shape
    n_chunks = S_PAD // _L
    n_workers = 32
    rows_per_w = R // n_workers
    b_per_w = rows_per_w // (na * K)
    mesh = plsc.VectorSubcoreMesh(core_axis_name="c", subcore_axis_name="s",
                                  num_cores=2, num_subcores=16)

    @functools.partial(
        pl.kernel,
        out_type=jax.ShapeDtypeStruct((K, nb, na, S_PAD), jnp.float32),
        mesh=mesh,
        compiler_params=pltpu.CompilerParams(needs_layout_passes=False),
        scratch_types=[
            pltpu.VMEM((rows_per_w * P,), jnp.float32),
            pltpu.VMEM((K, S_PAD), jnp.int32),
            pltpu.VMEM((K, S_PAD), jnp.float32),
            pltpu.VMEM((K, b_per_w, na, S_PAD), jnp.float32),
            pltpu.SemaphoreType.DMA,
        ],
    )
    def sc_k(y_hbm, fidx_hbm, fw_hbm, out_hbm, y_v, idx_v, w_v, out_v, sem):
        wid = lax.axis_index("s") * 2 + lax.axis_index("c")
        base = wid * rows_per_w
        pltpu.sync_copy(y_hbm.at[pl.ds(base * P, rows_per_w * P)], y_v)
        pltpu.sync_copy(fidx_hbm, idx_v)
        pltpu.sync_copy(fw_hbm, w_v)

        half = n_chunks // 2
        s_half = half * _L

        def make_body(lo):
            def body(j):
                col = j * _L
                iv = [idx_v[k, pl.ds(col, _L)] for k in range(K)]
                wv = [w_v[k, pl.ds(col, _L)] for k in range(K)]
                for r in range(rows_per_w):
                    t = r % 3
                    bl = (r // 3) // na
                    a = (r // 3) % na
                    acc = plsc.load_gather(y_v, [iv[0] + r * P]) * wv[0]
                    for k in range(1, K):
                        acc = acc + plsc.load_gather(y_v, [iv[k] + r * P]) * wv[k]
                    out_v[t, bl, a, pl.ds(col, _L)] = acc
            return body

        plsc.parallel_loop(0, half)(make_body(0))
        cps = [pltpu.async_copy(out_v.at[t, bl, :, pl.ds(0, s_half)],
                                out_hbm.at[t, wid * b_per_w + bl, :,
                                           pl.ds(0, s_half)], sem)
               for t in range(K) for bl in range(b_per_w)]
        plsc.parallel_loop(half, n_chunks)(make_body(0))
        cps += [pltpu.async_copy(out_v.at[t, bl, :, pl.ds(s_half, S_PAD - s_half)],
                                 out_hbm.at[t, wid * b_per_w + bl, :,
                                            pl.ds(s_half, S_PAD - s_half)], sem)
                for t in range(K) for bl in range(b_per_w)]
        for cp in cps:
            cp.wait()

    return sc_k(y2.reshape(R * P), fidx_p, fw_p)


def _tc_time_combine(hf4, time_idx, time_w, bb, nsc):
    T, NB, NA, S_PAD = hf4.shape
    K, O = time_idx.shape

    def tc_body(tidx_ref, tw_ref, hf_ref, out_ref):
        h = [hf_ref[t, :, :, :nsc] for t in range(T)]
        for o in range(O):
            c = []
            for t in range(T):
                ct = jnp.float32(0.0)
                for k in range(K):
                    ct = ct + jnp.where(tidx_ref[k, o] == t,
                                        tw_ref[k, o], jnp.float32(0.0))
                c.append(ct)
            acc = h[0] * c[0]
            for t in range(1, T):
                acc = acc + h[t] * c[t]
            out_ref[:, o, :, :] = acc

    return pl.pallas_call(
        tc_body,
        grid=(NB // bb,),
        in_specs=[
            pl.BlockSpec(memory_space=pltpu.SMEM),
            pl.BlockSpec(memory_space=pltpu.SMEM),
            pl.BlockSpec((T, bb, NA, S_PAD), lambda i: (0, i, 0, 0)),
        ],
        out_specs=pl.BlockSpec((bb, O, NA, nsc), lambda i: (i, 0, 0, 0)),
        out_shape=jax.ShapeDtypeStruct((NB, O, NA, nsc), jnp.float32),
    )(time_idx, time_w, hf4)


def kernel(y_pilots, freq_idx, freq_w, time_idx, time_w):
    B, A, T, P = y_pilots.shape
    K, NSC = freq_idx.shape
    O = time_idx.shape[1]
    S_PAD = ((NSC + 127) // 128) * 128
    R = B * A * T

    fidx_p = jnp.zeros((K, S_PAD), jnp.int32).at[:, :NSC].set(freq_idx)
    fw_p = jnp.zeros((K, S_PAD), jnp.float32).at[:, :NSC].set(freq_w)

    hf4 = _sc_freq_interp(y_pilots.reshape(R, P), fidx_p, fw_p, B, A)
    out4 = _tc_time_combine(hf4, time_idx, time_w, bb=16, nsc=NSC)
    return jnp.transpose(out4, (0, 2, 1, 3))
